# baseline (speedup 1.0000x reference)
import functools

import jax
import jax.numpy as jnp
from jax import lax
from jax.experimental import pallas as pl
from jax.experimental.pallas import tpu as pltpu
from jax.experimental.pallas import tpu_sc as plsc

_NC = 2
_NS = 16
_SL = 4


@functools.lru_cache(maxsize=None)
def _build(B, D, K):
    assert D % _NC == 0
    assert B % _NS == 0
    dpc = D // _NC
    bpt = B // _NS

    mesh = plsc.VectorSubcoreMesh(
        core_axis_name="c", subcore_axis_name="s",
        num_cores=_NC, num_subcores=_NS,
    )

    @functools.partial(
        pl.kernel,
        out_type=(
            jax.ShapeDtypeStruct((D, B), jnp.float32),
            jax.ShapeDtypeStruct((D, B), jnp.float32),
        ),
        mesh=mesh,
        scratch_types=[
            [pltpu.VMEM_SHARED((K,), jnp.float32) for _ in range(_SL)],
            [pltpu.VMEM_SHARED((K,), jnp.float32) for _ in range(_SL)],
            [pltpu.VMEM((bpt,), jnp.int32) for _ in range(dpc)],
            [pltpu.VMEM((bpt,), jnp.float32) for _ in range(2)],
            [pltpu.VMEM((bpt,), jnp.float32) for _ in range(2)],
            pltpu.SemaphoreType.DMA,
            [pltpu.SemaphoreType.DMA for _ in range(_SL)],
            [pltpu.SemaphoreType.DMA for _ in range(_SL)],
            [pltpu.SemaphoreType.DMA for _ in range(2)],
            [pltpu.SemaphoreType.DMA for _ in range(2)],
            [pltpu.SemaphoreType.DMA for _ in range(2)],
            [pltpu.SemaphoreType.DMA for _ in range(2)],
        ],
    )
    def gather_kernel(labt_hbm, mean_hbm, lv_hbm, outm_hbm, outv_hbm,
                      sm_slots, sv_slots, lab_vs, gm_vs, gv_vs,
                      sem_lab, sems_sm, sems_sv, sems_gm, sems_gv,
                      sems_wm, sems_wv):
        c = lax.axis_index("c")
        s = lax.axis_index("s")
        d0 = c * dpc

        lab_cps = []
        for dd in range(dpc):
            lab_cps.append(pltpu.async_copy(
                labt_hbm.at[d0 + dd, pl.ds(s * bpt, bpt)],
                lab_vs[dd], sem_lab))

        def stage(dd):
            sl = dd % _SL

            @pl.when(s == (2 * dd) % _NS)
            def _():
                pltpu.async_copy(
                    mean_hbm.at[d0 + dd], sm_slots[sl], sems_sm[sl])

            @pl.when(s == (2 * dd + 1) % _NS)
            def _():
                pltpu.async_copy(
                    lv_hbm.at[d0 + dd], sv_slots[sl], sems_sv[sl])

        def stage_wait(dd):
            sl = dd % _SL

            @pl.when(s == (2 * dd) % _NS)
            def _():
                pltpu.make_async_copy(
                    mean_hbm.at[d0 + dd], sm_slots[sl], sems_sm[sl]).wait()

            @pl.when(s == (2 * dd + 1) % _NS)
            def _():
                pltpu.make_async_copy(
                    lv_hbm.at[d0 + dd], sv_slots[sl], sems_sv[sl]).wait()

        for dd in range(min(_SL - 1, dpc)):
            stage(dd)

        for cp in lab_cps:
            cp.wait()

        def writeback(dd):
            pr = dd % 2
            out_slice = pl.ds(s * bpt, bpt)
            wm = pltpu.async_copy(
                gm_vs[pr], outm_hbm.at[d0 + dd, out_slice], sems_wm[pr])
            wv = pltpu.async_copy(
                gv_vs[pr], outv_hbm.at[d0 + dd, out_slice], sems_wv[pr])
            return wm, wv

        wbs = {}
        prev = None
        for dd in range(dpc):
            sl = dd % _SL
            pr = dd % 2
            stage_wait(dd)
            if prev is not None:
                pm, pv = prev
                pm.wait()
                pv.wait()
            if dd >= 2:
                wm, wv = wbs.pop(dd - 2)
                wm.wait()
                wv.wait()
            plsc.subcore_barrier()
            if dd + _SL - 1 < dpc:
                stage(dd + _SL - 1)
            cp_m = pltpu.async_copy(
                sm_slots[sl].at[lab_vs[dd]], gm_vs[pr], sems_gm[pr])
            cp_v = pltpu.async_copy(
                sv_slots[sl].at[lab_vs[dd]], gv_vs[pr], sems_gv[pr])
            if dd >= 1:
                wbs[dd - 1] = writeback(dd - 1)
            prev = (cp_m, cp_v)

        pm, pv = prev
        pm.wait()
        pv.wait()
        wbs[dpc - 1] = writeback(dpc - 1)
        for dd in sorted(wbs):
            wm, wv = wbs[dd]
            wm.wait()
            wv.wait()

    return gather_kernel


def kernel(labels, mean, log_var):
    B, D = labels.shape
    K = mean.shape[1]
    gk = _build(B, D, K)
    labt = jnp.transpose(labels.astype(jnp.int32))
    outm_t, outv_t = gk(labt, mean, log_var)
    return jnp.transpose(outm_t), jnp.transpose(outv_t)
